# Initial kernel scaffold; baseline (speedup 1.0000x reference)
import jax
import jax.numpy as jnp
from jax.experimental import pallas as pl


def kernel(x, positions, pe, ln_w, ln_b):
    raise NotImplementedError("write your pallas kernel here")


# speedup vs baseline: 1.4692x; 1.4692x over previous
import functools
import math

import jax
import jax.numpy as jnp
from jax import lax
from jax.experimental import pallas as pl
from jax.experimental.pallas import tpu as pltpu
from jax.experimental.pallas import tpu_sc as plsc

B, T, D, MAX_LEN = 4, 8192, 768, 8192
EPS = 1e-5
SCALE = math.sqrt(float(D))
N = B * T

ROWS_BLK = 512


def _ln_body(pe_ref, w_ref, b_ref, out_ref):
    y = pe_ref[...] * SCALE
    mu = jnp.mean(y, axis=-1, keepdims=True)
    yc = y - mu
    var = jnp.mean(yc * yc, axis=-1, keepdims=True)
    out_ref[...] = yc * lax.rsqrt(var + EPS) * w_ref[...] + b_ref[...]


def _normalize_table(pe, ln_w, ln_b):
    return pl.pallas_call(
        _ln_body,
        grid=(MAX_LEN // ROWS_BLK,),
        in_specs=[
            pl.BlockSpec((ROWS_BLK, D), lambda i: (i, 0)),
            pl.BlockSpec((1, D), lambda i: (0, 0)),
            pl.BlockSpec((1, D), lambda i: (0, 0)),
        ],
        out_specs=pl.BlockSpec((ROWS_BLK, D), lambda i: (i, 0)),
        out_shape=jax.ShapeDtypeStruct((MAX_LEN, D), jnp.float32),
    )(pe, ln_w.reshape(1, D), ln_b.reshape(1, D))


_NC, _NS = 2, 16
NW = _NC * _NS
RPW = N // NW
CHUNK = 64
NCHUNK = RPW // CHUNK
LPR = D // 16


@functools.cache
def _make_gather_add():
    mesh = plsc.VectorSubcoreMesh(core_axis_name="c", subcore_axis_name="s",
                                  num_cores=_NC, num_subcores=_NS)

    @functools.partial(
        pl.kernel,
        out_type=jax.ShapeDtypeStruct((N, D), jnp.float32),
        mesh=mesh,
        scratch_types=[
            pltpu.VMEM((CHUNK,), jnp.int32),
            pltpu.VMEM((CHUNK, D), jnp.float32),
            pltpu.VMEM((CHUNK, D), jnp.float32),
            pltpu.SemaphoreType.DMA,
            pltpu.SemaphoreType.DMA,
        ],
    )
    def gather_add(table_hbm, idx_hbm, x_hbm, out_hbm,
                   idx_v, xb, gb, sem_x, sem_g):
        wid = lax.axis_index("s") * _NC + lax.axis_index("c")
        base = pl.multiple_of(wid * RPW, CHUNK)

        def step(k, carry):
            off = pl.multiple_of(base + k * CHUNK, CHUNK)
            pltpu.sync_copy(idx_hbm.at[pl.ds(off, CHUNK)], idx_v)
            cg = pltpu.async_copy(table_hbm.at[idx_v], gb, sem_g)
            cx = pltpu.async_copy(x_hbm.at[pl.ds(off, CHUNK)], xb, sem_x)
            cg.wait()
            cx.wait()

            def row(i, c):
                for j in range(LPR):
                    sl = pl.ds(j * 16, 16)
                    xb[i, sl] = xb[i, sl] + gb[i, sl]
                return c

            lax.fori_loop(0, CHUNK, row, 0)
            pltpu.sync_copy(xb, out_hbm.at[pl.ds(off, CHUNK)])
            return carry

        lax.fori_loop(0, NCHUNK, step, 0)

    return gather_add


def kernel(x, positions, pe, ln_w, ln_b):
    pe_norm = _normalize_table(pe, ln_w, ln_b)
    out = _make_gather_add()(pe_norm, positions.reshape(N).astype(jnp.int32),
                             x.reshape(N, D))
    return out.reshape(B, T, D)
